# baseline (speedup 1.0000x reference)
import functools

import jax
import jax.numpy as jnp
from jax import lax
from jax.experimental import pallas as pl
from jax.experimental.pallas import tpu as pltpu
from jax.experimental.pallas import tpu_sc as plsc

_LANES = 16


def _build_sc_kernel(b: int, s: int, n_idx: int):
    n_flat = b * s
    info = plsc.get_sparse_core_info()
    num_cores, num_subcores = 1, 4
    n_workers = num_cores * num_subcores
    per_worker = n_flat // n_workers
    slices_per_row = s // per_worker

    mesh = plsc.VectorSubcoreMesh(
        core_axis_name="c", subcore_axis_name="s",
        num_cores=num_cores, num_subcores=num_subcores,
    )

    @functools.partial(
        pl.kernel,
        mesh=mesh,
        out_type=jax.ShapeDtypeStruct((b, s), jnp.int32),
        scratch_types=[
            pltpu.VMEM((n_idx,), jnp.int32),
            pltpu.VMEM((per_worker,), jnp.int32),
            pltpu.SemaphoreType.DMA,
        ],
        compiler_params=pltpu.CompilerParams(
            needs_layout_passes=False,
            skip_device_barrier=True,
            disable_bounds_checks=True,
            disable_semaphore_checks=True,
        ),
    )
    def sc_kern(idx_hbm, out_hbm, idx_v, buf_v, sem):
        wid = lax.axis_index("s") * num_cores + lax.axis_index("c")
        base = wid * per_worker

        idx_copy = pltpu.async_copy(idx_hbm, idx_v, sem)

        zeros = jnp.zeros((_LANES,), jnp.int32)
        ones = jnp.ones((_LANES,), jnp.int32)

        @plsc.parallel_loop(0, per_worker // _LANES, unroll=8)
        def _zero(i):
            buf_v[pl.ds(i * _LANES, _LANES)] = zeros

        idx_copy.wait()

        @plsc.parallel_loop(0, n_idx // _LANES, unroll=8)
        def _scatter(i):
            v = idx_v[pl.ds(i * _LANES, _LANES)]
            off = v - base
            in_range = off.astype(jnp.uint32) < jnp.uint32(per_worker)
            plsc.store_scatter(
                buf_v, [off & (per_worker - 1)], ones, mask=in_range
            )

        row = wid // slices_per_row
        col = (wid % slices_per_row) * per_worker
        pltpu.sync_copy(buf_v, out_hbm.at[row, pl.ds(col, per_worker)])

    return sc_kern


def kernel(vision_indices, reference_tensor):
    b, s = reference_tensor.shape[0], reference_tensor.shape[1]
    idx32 = vision_indices.astype(jnp.int32)
    mask = _build_sc_kernel(b, s, idx32.shape[0])(idx32)
    return mask.astype(bool)


# speedup vs baseline: 1.0022x; 1.0022x over previous
import functools

import jax
import jax.numpy as jnp
from jax import lax
from jax.experimental import pallas as pl
from jax.experimental.pallas import tpu as pltpu
from jax.experimental.pallas import tpu_sc as plsc

_LANES = 16


def _build_sc_kernel(b: int, s: int, n_idx: int):
    n_flat = b * s
    info = plsc.get_sparse_core_info()
    num_cores, num_subcores = 1, 8
    n_workers = num_cores * num_subcores
    per_worker = n_flat // n_workers
    slices_per_row = s // per_worker

    mesh = plsc.VectorSubcoreMesh(
        core_axis_name="c", subcore_axis_name="s",
        num_cores=num_cores, num_subcores=num_subcores,
    )

    @functools.partial(
        pl.kernel,
        mesh=mesh,
        out_type=jax.ShapeDtypeStruct((b, s), jnp.int32),
        scratch_types=[
            pltpu.VMEM((n_idx,), jnp.int32),
            pltpu.VMEM((per_worker,), jnp.int32),
            pltpu.SemaphoreType.DMA,
        ],
        compiler_params=pltpu.CompilerParams(
            needs_layout_passes=False,
            skip_device_barrier=True,
            disable_bounds_checks=True,
            disable_semaphore_checks=True,
        ),
    )
    def sc_kern(idx_hbm, out_hbm, idx_v, buf_v, sem):
        wid = lax.axis_index("s") * num_cores + lax.axis_index("c")
        base = wid * per_worker

        idx_copy = pltpu.async_copy(idx_hbm, idx_v, sem)

        zeros = jnp.zeros((_LANES,), jnp.int32)
        ones = jnp.ones((_LANES,), jnp.int32)

        @plsc.parallel_loop(0, per_worker // _LANES, unroll=8)
        def _zero(i):
            buf_v[pl.ds(i * _LANES, _LANES)] = zeros

        idx_copy.wait()

        @plsc.parallel_loop(0, n_idx // _LANES, unroll=8)
        def _scatter(i):
            v = idx_v[pl.ds(i * _LANES, _LANES)]
            off = v - base
            in_range = off.astype(jnp.uint32) < jnp.uint32(per_worker)
            plsc.store_scatter(
                buf_v, [off & (per_worker - 1)], ones, mask=in_range
            )

        row = wid // slices_per_row
        col = (wid % slices_per_row) * per_worker
        pltpu.sync_copy(buf_v, out_hbm.at[row, pl.ds(col, per_worker)])

    return sc_kern


def kernel(vision_indices, reference_tensor):
    b, s = reference_tensor.shape[0], reference_tensor.shape[1]
    idx32 = vision_indices.astype(jnp.int32)
    mask = _build_sc_kernel(b, s, idx32.shape[0])(idx32)
    return mask.astype(bool)
